# baseline (speedup 1.0000x reference)
import functools

import jax
import jax.numpy as jnp
from jax import lax
from jax.experimental import pallas as pl
from jax.experimental.pallas import tpu as pltpu
from jax.experimental.pallas import tpu_sc as plsc

BATCH = 4
SEQ = 8192
D = 1024
R = 16
NW = 32
ROWS_PER_W = SEQ // NW
N_CHUNKS = ROWS_PER_W // R
LANES = 16


def _sc_body(x_hbm, pos_hbm, out_hbm, x_v, pos_v):
    wid = lax.axis_index("s") * 2 + lax.axis_index("c")
    s0 = wid * ROWS_PER_W

    def chunk(c, carry):
        row = s0 + c * R
        pltpu.sync_copy(pos_hbm.at[pl.ds(row, R), :], pos_v)
        for b in range(BATCH):
            pltpu.sync_copy(x_hbm.at[b, pl.ds(row, R), :], x_v.at[b])

        def col(i, carry2):
            cs = i * LANES
            for r in range(R):
                p = pos_v[r, pl.ds(cs, LANES)]
                for b in range(BATCH):
                    x_v[b, r, pl.ds(cs, LANES)] = x_v[b, r, pl.ds(cs, LANES)] + p
            return carry2

        lax.fori_loop(0, D // LANES, col, 0)
        for b in range(BATCH):
            pltpu.sync_copy(x_v.at[b], out_hbm.at[b, pl.ds(row, R), :])
        return carry

    lax.fori_loop(0, N_CHUNKS, chunk, 0)


def kernel(x, pos_embedding):
    batch, seq_len, d_model = x.shape
    mesh = plsc.VectorSubcoreMesh(core_axis_name="c", subcore_axis_name="s")
    f = functools.partial(
        pl.kernel,
        out_type=jax.ShapeDtypeStruct((batch, seq_len, d_model), x.dtype),
        mesh=mesh,
        scratch_types=[
            pltpu.VMEM((BATCH, R, D), jnp.float32),
            pltpu.VMEM((R, D), jnp.float32),
        ],
    )(_sc_body)
    return f(x, pos_embedding[:seq_len])


# speedup vs baseline: 1.7815x; 1.7815x over previous
import functools

import jax
import jax.numpy as jnp
from jax import lax
from jax.experimental import pallas as pl
from jax.experimental.pallas import tpu as pltpu
from jax.experimental.pallas import tpu_sc as plsc

BATCH = 4
SEQ = 8192
D = 1024
R = 8
NW = 32
ROWS_PER_W = SEQ // NW
N_CHUNKS = ROWS_PER_W // R
LANES = 16


def _sc_body(x_hbm, pos_hbm, out_hbm, x_v, pos_v, sin0, sin1, sout0, sout1):
    sin = (sin0, sin1)
    sout = (sout0, sout1)
    wid = lax.axis_index("s") * 2 + lax.axis_index("c")
    s0 = wid * ROWS_PER_W

    def start_in(c, k):
        row = s0 + c * R
        pltpu.make_async_copy(pos_hbm.at[pl.ds(row, R), :], pos_v.at[k], sin[k]).start()
        for b in range(BATCH):
            pltpu.make_async_copy(x_hbm.at[b, pl.ds(row, R), :], x_v.at[k, b], sin[k]).start()

    def wait_in(k):
        pltpu.make_async_copy(pos_hbm.at[pl.ds(0, R), :], pos_v.at[k], sin[k]).wait()
        for b in range(BATCH):
            pltpu.make_async_copy(x_hbm.at[b, pl.ds(0, R), :], x_v.at[k, b], sin[k]).wait()

    def start_out(c, k):
        row = s0 + c * R
        for b in range(BATCH):
            pltpu.make_async_copy(x_v.at[k, b], out_hbm.at[b, pl.ds(row, R), :], sout[k]).start()

    def wait_out(k):
        for b in range(BATCH):
            pltpu.make_async_copy(x_v.at[k, b], out_hbm.at[b, pl.ds(0, R), :], sout[k]).wait()

    def compute(k):
        def col(i, carry):
            cs = i * LANES
            for r in range(R):
                p = pos_v[k, r, pl.ds(cs, LANES)]
                for b in range(BATCH):
                    x_v[k, b, r, pl.ds(cs, LANES)] = x_v[k, b, r, pl.ds(cs, LANES)] + p
            return carry

        lax.fori_loop(0, D // LANES, col, 0)

    start_in(0, 0)

    def pair(cc, carry):
        for kk in range(2):
            c = cc * 2 + kk
            other = (kk + 1) % 2

            @pl.when(c > 0)
            def _():
                wait_out(other)

            @pl.when(c + 1 < N_CHUNKS)
            def _():
                start_in(c + 1, other)

            wait_in(kk)
            compute(kk)
            start_out(c, kk)
        return carry

    lax.fori_loop(0, N_CHUNKS // 2, pair, 0)
    wait_out(1)


def kernel(x, pos_embedding):
    batch, seq_len, d_model = x.shape
    mesh = plsc.VectorSubcoreMesh(core_axis_name="c", subcore_axis_name="s")
    f = functools.partial(
        pl.kernel,
        out_type=jax.ShapeDtypeStruct((batch, seq_len, d_model), x.dtype),
        mesh=mesh,
        scratch_types=[
            pltpu.VMEM((2, BATCH, R, D), jnp.float32),
            pltpu.VMEM((2, R, D), jnp.float32),
            pltpu.SemaphoreType.DMA,
            pltpu.SemaphoreType.DMA,
            pltpu.SemaphoreType.DMA,
            pltpu.SemaphoreType.DMA,
        ],
    )(_sc_body)
    return f(x, pos_embedding[:seq_len])


# speedup vs baseline: 1.8706x; 1.0500x over previous
import functools

import jax
import jax.numpy as jnp
from jax import lax
from jax.experimental import pallas as pl
from jax.experimental.pallas import tpu as pltpu
from jax.experimental.pallas import tpu_sc as plsc

BATCH = 4
SEQ = 8192
D = 1024
R = 8
NW = 32
ROWS_PER_W = SEQ // NW
N_CHUNKS = ROWS_PER_W // R
LANES = 16


def _sc_body(x_hbm, pos_hbm, out_hbm, x_v, pos_v, sin0, sin1, sout0, sout1):
    sin = (sin0, sin1)
    sout = (sout0, sout1)
    wid = lax.axis_index("s") * 2 + lax.axis_index("c")
    s0 = wid * ROWS_PER_W

    def start_in(c, k):
        row = s0 + c * R
        pltpu.make_async_copy(pos_hbm.at[pl.ds(row, R), :], pos_v.at[k], sin[k]).start()
        for b in range(BATCH):
            pltpu.make_async_copy(x_hbm.at[b, pl.ds(row, R), :], x_v.at[k, b], sin[k]).start()

    def wait_in(k):
        pltpu.make_async_copy(pos_hbm.at[pl.ds(0, R), :], pos_v.at[k], sin[k]).wait()
        for b in range(BATCH):
            pltpu.make_async_copy(x_hbm.at[b, pl.ds(0, R), :], x_v.at[k, b], sin[k]).wait()

    def start_out(c, k):
        row = s0 + c * R
        for b in range(BATCH):
            pltpu.make_async_copy(x_v.at[k, b], out_hbm.at[b, pl.ds(row, R), :], sout[k]).start()

    def wait_out(k):
        for b in range(BATCH):
            pltpu.make_async_copy(x_v.at[k, b], out_hbm.at[b, pl.ds(0, R), :], sout[k]).wait()

    def compute(k):
        @plsc.parallel_loop(0, D // LANES, carry=jnp.int32(0))
        def col(i, carry):
            cs = i * LANES
            for r in range(R):
                p = pos_v[k, r, pl.ds(cs, LANES)]
                for b in range(BATCH):
                    x_v[k, b, r, pl.ds(cs, LANES)] = x_v[k, b, r, pl.ds(cs, LANES)] + p
            return carry

    start_in(0, 0)

    def pair(cc, carry):
        for kk in range(2):
            c = cc * 2 + kk
            other = (kk + 1) % 2

            @pl.when(c > 0)
            def _():
                wait_out(other)

            @pl.when(c + 1 < N_CHUNKS)
            def _():
                start_in(c + 1, other)

            wait_in(kk)
            compute(kk)
            start_out(c, kk)
        return carry

    lax.fori_loop(0, N_CHUNKS // 2, pair, 0)
    wait_out(1)


def kernel(x, pos_embedding):
    batch, seq_len, d_model = x.shape
    mesh = plsc.VectorSubcoreMesh(core_axis_name="c", subcore_axis_name="s")
    f = functools.partial(
        pl.kernel,
        out_type=jax.ShapeDtypeStruct((batch, seq_len, d_model), x.dtype),
        mesh=mesh,
        scratch_types=[
            pltpu.VMEM((2, BATCH, R, D), jnp.float32),
            pltpu.VMEM((2, R, D), jnp.float32),
            pltpu.SemaphoreType.DMA,
            pltpu.SemaphoreType.DMA,
            pltpu.SemaphoreType.DMA,
            pltpu.SemaphoreType.DMA,
        ],
    )(_sc_body)
    return f(x, pos_embedding[:seq_len])


# speedup vs baseline: 1.9704x; 1.0533x over previous
import functools

import jax
import jax.numpy as jnp
from jax import lax
from jax.experimental import pallas as pl
from jax.experimental.pallas import tpu as pltpu
from jax.experimental.pallas import tpu_sc as plsc

BATCH = 4
SEQ = 8192
D = 1024
LANES = 16

SC_ROWS = 2048
TC_ROWS = SEQ - SC_ROWS

R = 8
NW = 32
ROWS_PER_W = SC_ROWS // NW
N_CHUNKS = ROWS_PER_W // R

S_BLK = 512


def _sc_body(x_hbm, pos_hbm, out_hbm, x_v, pos_v, sin0, sin1, sout0, sout1):
    sin = (sin0, sin1)
    sout = (sout0, sout1)
    wid = lax.axis_index("s") * 2 + lax.axis_index("c")
    s0 = TC_ROWS + wid * ROWS_PER_W

    def start_in(c, k):
        row = s0 + c * R
        pltpu.make_async_copy(pos_hbm.at[pl.ds(row, R), :], pos_v.at[k], sin[k]).start()
        for b in range(BATCH):
            pltpu.make_async_copy(x_hbm.at[b, pl.ds(row, R), :], x_v.at[k, b], sin[k]).start()

    def wait_in(k):
        pltpu.make_async_copy(pos_hbm.at[pl.ds(0, R), :], pos_v.at[k], sin[k]).wait()
        for b in range(BATCH):
            pltpu.make_async_copy(x_hbm.at[b, pl.ds(0, R), :], x_v.at[k, b], sin[k]).wait()

    def start_out(c, k):
        row = s0 + c * R
        for b in range(BATCH):
            pltpu.make_async_copy(x_v.at[k, b], out_hbm.at[b, pl.ds(row, R), :], sout[k]).start()

    def wait_out(k):
        for b in range(BATCH):
            pltpu.make_async_copy(x_v.at[k, b], out_hbm.at[b, pl.ds(0, R), :], sout[k]).wait()

    def compute(k):
        @plsc.parallel_loop(0, D // LANES, carry=jnp.int32(0))
        def col(i, carry):
            cs = i * LANES
            for r in range(R):
                p = pos_v[k, r, pl.ds(cs, LANES)]
                for b in range(BATCH):
                    x_v[k, b, r, pl.ds(cs, LANES)] = x_v[k, b, r, pl.ds(cs, LANES)] + p
            return carry

    start_in(0, 0)

    def pair(cc, carry):
        for kk in range(2):
            c = cc * 2 + kk
            other = (kk + 1) % 2

            @pl.when(c > 0)
            def _():
                wait_out(other)

            @pl.when(c + 1 < N_CHUNKS)
            def _():
                start_in(c + 1, other)

            wait_in(kk)
            compute(kk)
            start_out(c, kk)
        return carry

    lax.fori_loop(0, N_CHUNKS // 2, pair, 0)
    wait_out(1)


def _tc_body(buf_ref, x_ref, pos_ref, out_ref):
    out_ref[...] = x_ref[...] + pos_ref[...][None, :, :]


def kernel(x, pos_embedding):
    batch, seq_len, d_model = x.shape
    pos = pos_embedding[:seq_len]

    mesh = plsc.VectorSubcoreMesh(core_axis_name="c", subcore_axis_name="s")
    sc_out = functools.partial(
        pl.kernel,
        out_type=jax.ShapeDtypeStruct((batch, seq_len, d_model), x.dtype),
        mesh=mesh,
        scratch_types=[
            pltpu.VMEM((2, BATCH, R, D), jnp.float32),
            pltpu.VMEM((2, R, D), jnp.float32),
            pltpu.SemaphoreType.DMA,
            pltpu.SemaphoreType.DMA,
            pltpu.SemaphoreType.DMA,
            pltpu.SemaphoreType.DMA,
        ],
    )(_sc_body)(x, pos)

    n_tc = TC_ROWS // S_BLK
    return pl.pallas_call(
        _tc_body,
        grid=(n_tc,),
        in_specs=[
            pl.BlockSpec(memory_space=pl.ANY),
            pl.BlockSpec((batch, S_BLK, d_model), lambda s: (0, s, 0)),
            pl.BlockSpec((S_BLK, d_model), lambda s: (s, 0)),
        ],
        out_specs=pl.BlockSpec((batch, S_BLK, d_model), lambda s: (0, s, 0)),
        out_shape=jax.ShapeDtypeStruct((batch, seq_len, d_model), x.dtype),
        input_output_aliases={0: 0},
    )(sc_out, x, pos)


# speedup vs baseline: 1.9766x; 1.0032x over previous
import functools

import jax
import jax.numpy as jnp
from jax import lax
from jax.experimental import pallas as pl
from jax.experimental.pallas import tpu as pltpu
from jax.experimental.pallas import tpu_sc as plsc

BATCH = 4
SEQ = 8192
D = 1024
LANES = 16

SC_ROWS = 2048
TC_ROWS = SEQ - SC_ROWS

R = 8
NW = 32
ROWS_PER_W = SC_ROWS // NW
N_CHUNKS = ROWS_PER_W // R

S_BLK = 512


def _sc_body(x_hbm, pos_hbm, out_hbm, x_v, pos_v, sin0, sin1, sout0, sout1):
    sin = (sin0, sin1)
    sout = (sout0, sout1)
    wid = lax.axis_index("s") * 2 + lax.axis_index("c")
    s0 = TC_ROWS + wid * ROWS_PER_W

    def start_in(c, k):
        row = s0 + c * R
        pltpu.make_async_copy(pos_hbm.at[pl.ds(row, R), :], pos_v.at[k], sin[k]).start()
        pltpu.make_async_copy(x_hbm.at[:, pl.ds(row, R), :], x_v.at[k], sin[k]).start()

    def wait_in(k):
        pltpu.make_async_copy(pos_hbm.at[pl.ds(0, R), :], pos_v.at[k], sin[k]).wait()
        pltpu.make_async_copy(x_hbm.at[:, pl.ds(0, R), :], x_v.at[k], sin[k]).wait()

    def start_out(c, k):
        row = s0 + c * R
        pltpu.make_async_copy(x_v.at[k], out_hbm.at[:, pl.ds(row, R), :], sout[k]).start()

    def wait_out(k):
        pltpu.make_async_copy(x_v.at[k], out_hbm.at[:, pl.ds(0, R), :], sout[k]).wait()

    def compute(k):
        @plsc.parallel_loop(0, D // LANES, carry=jnp.int32(0))
        def col(i, carry):
            cs = i * LANES
            for r in range(R):
                p = pos_v[k, r, pl.ds(cs, LANES)]
                for b in range(BATCH):
                    x_v[k, b, r, pl.ds(cs, LANES)] = x_v[k, b, r, pl.ds(cs, LANES)] + p
            return carry

    start_in(0, 0)

    def pair(cc, carry):
        for kk in range(2):
            c = cc * 2 + kk
            other = (kk + 1) % 2

            @pl.when(c > 0)
            def _():
                wait_out(other)

            @pl.when(c + 1 < N_CHUNKS)
            def _():
                start_in(c + 1, other)

            wait_in(kk)
            compute(kk)
            start_out(c, kk)
        return carry

    lax.fori_loop(0, N_CHUNKS // 2, pair, 0)
    wait_out(1)


def _tc_body(buf_ref, x_ref, pos_ref, out_ref):
    out_ref[...] = x_ref[...] + pos_ref[...][None, :, :]


def kernel(x, pos_embedding):
    batch, seq_len, d_model = x.shape
    pos = pos_embedding[:seq_len]

    mesh = plsc.VectorSubcoreMesh(core_axis_name="c", subcore_axis_name="s")
    sc_out = functools.partial(
        pl.kernel,
        out_type=jax.ShapeDtypeStruct((batch, seq_len, d_model), x.dtype),
        mesh=mesh,
        scratch_types=[
            pltpu.VMEM((2, BATCH, R, D), jnp.float32),
            pltpu.VMEM((2, R, D), jnp.float32),
            pltpu.SemaphoreType.DMA,
            pltpu.SemaphoreType.DMA,
            pltpu.SemaphoreType.DMA,
            pltpu.SemaphoreType.DMA,
        ],
    )(_sc_body)(x, pos)

    n_tc = TC_ROWS // S_BLK
    return pl.pallas_call(
        _tc_body,
        grid=(n_tc,),
        in_specs=[
            pl.BlockSpec(memory_space=pl.ANY),
            pl.BlockSpec((batch, S_BLK, d_model), lambda s: (0, s, 0)),
            pl.BlockSpec((S_BLK, d_model), lambda s: (s, 0)),
        ],
        out_specs=pl.BlockSpec((batch, S_BLK, d_model), lambda s: (0, s, 0)),
        out_shape=jax.ShapeDtypeStruct((batch, seq_len, d_model), x.dtype),
        input_output_aliases={0: 0},
    )(sc_out, x, pos)
